# baseline (speedup 1.0000x reference)
import functools

import jax
import jax.numpy as jnp
from jax import lax
from jax.experimental import pallas as pl
from jax.experimental.pallas import tpu as pltpu
from jax.experimental.pallas import tpu_sc as plsc

BUFFER_SIZE = 10000
N_CLASSES = 100
BATCH = 4096
ROW = 3 * 32 * 32

R1 = BATCH * ROW
R2 = (BUFFER_SIZE - BATCH) * ROW
TOT = BUFFER_SIZE * ROW

NTILES = 32
S1 = R1 // NTILES
S2 = R2 // NTILES

CHUNK = 12_288
NB = 8
DEPTH = 4


def _body(x_h, y_h, t_h, bx_h, by_h, bt_h, obx_h, oby_h, obt_h,
          spbuf, tfill, tailb, tvec, sin, sout):
    cid = lax.axis_index("c")
    sid = lax.axis_index("s")
    wid = sid * 2 + cid

    chunks = []
    for k in range(S1 // CHUNK):
        chunks.append((x_h, wid * S1 + k * CHUNK, CHUNK))
    base2 = R1 + wid * S2
    n2, rem = divmod(S2, CHUNK)
    for k in range(n2):
        chunks.append((bx_h, base2 + k * CHUNK, CHUNK))
    if rem:
        chunks.append((bx_h, base2 + n2 * CHUNK, rem))
    n = len(chunks)

    in_d = [None] * n
    out_d = [None] * n

    def start_in(i):
        src, off, sz = chunks[i]
        b = i % NB
        in_d[i] = pltpu.async_copy(
            src.at[pl.ds(off, sz)],
            spbuf.at[sid, b, pl.ds(0, sz)], sin.at[b])

    for i in range(min(DEPTH, n)):
        start_in(i)
    for i in range(n):
        _, off, sz = chunks[i]
        b = i % NB
        in_d[i].wait()
        out_d[i] = pltpu.async_copy(
            spbuf.at[sid, b, pl.ds(0, sz)],
            obx_h.at[pl.ds(off, sz)], sout.at[b])
        j = i + DEPTH
        if j < n:
            if j >= NB:
                out_d[j - NB].wait()
            start_in(j)
    for i in range(max(0, n - NB), n):
        out_d[i].wait()

    TAIL = BUFFER_SIZE - BATCH

    @pl.when(wid == 30)
    def _():
        pltpu.sync_copy(y_h, tfill)
        pltpu.sync_copy(tfill, oby_h.at[pl.ds(0, BATCH)])
        pltpu.sync_copy(by_h.at[pl.ds(BATCH, TAIL)], tailb)
        pltpu.sync_copy(tailb, oby_h.at[pl.ds(BATCH, TAIL)])

    @pl.when(wid == 31)
    def _():
        pltpu.sync_copy(t_h, tvec)
        tv = tvec[...]
        for i in range(BATCH // 16):
            tfill[pl.ds(i * 16, 16)] = tv
        pltpu.sync_copy(tfill, obt_h.at[pl.ds(0, BATCH)])
        pltpu.sync_copy(bt_h.at[pl.ds(BATCH, TAIL)], tailb)
        pltpu.sync_copy(tailb, obt_h.at[pl.ds(BATCH, TAIL)])


@jax.jit
def _er_update(x, y, t16, bx, by, bt):
    xf = x.reshape(R1)
    bxf = bx.reshape(TOT)
    mesh = plsc.VectorSubcoreMesh(core_axis_name="c", subcore_axis_name="s")
    run = pl.kernel(
        _body,
        out_type=(
            jax.ShapeDtypeStruct((TOT,), jnp.float32),
            jax.ShapeDtypeStruct((BUFFER_SIZE,), jnp.int32),
            jax.ShapeDtypeStruct((BUFFER_SIZE,), jnp.int32),
        ),
        mesh=mesh,
        scratch_types=[
            pltpu.VMEM_SHARED((16, NB, CHUNK), jnp.float32),
            pltpu.VMEM((BATCH,), jnp.int32),
            pltpu.VMEM((BUFFER_SIZE - BATCH,), jnp.int32),
            pltpu.VMEM((16,), jnp.int32),
            pltpu.SemaphoreType.DMA((NB,)),
            pltpu.SemaphoreType.DMA((NB,)),
        ],
    )
    obx, oby, obt = run(xf, y, t16, bxf, by, bt)
    return obx.reshape(bx.shape), oby, obt


def kernel(x, y, task_id, bx, by, bt):
    t16 = jnp.full((16,), task_id, dtype=jnp.int32)
    return _er_update(x, y, t16, bx, by, bt)
